# baseline (speedup 1.0000x reference)
import functools
import itertools

import jax
import jax.numpy as jnp
from jax import lax
from jax.experimental import pallas as pl
from jax.experimental.pallas import tpu as pltpu
from jax.experimental.pallas import tpu_sc as plsc

_MAX_SPIKE = 100.0
_P = 0.1
_NC, _NS = 2, 16
_NW = _NC * _NS
_CW = 2048
_DT = 3
_DS = 2


def _spike_consts(B, T, dtype):
    key = jax.random.key(42)
    k1, k2, k3 = jax.random.split(key, 3)
    probas = jax.random.uniform(k1, (B,), dtype=jnp.float32)
    mask = probas > (1.0 - _P)
    pos = jax.random.randint(k2, (B,), 0, T - 2)
    sign = jnp.where(jax.random.randint(k3, (), 0, 2) == 0, -1.0, 1.0).astype(dtype)
    vals = jnp.where(mask, sign * _MAX_SPIKE, 0.0).astype(dtype)
    return pos, vals


def _sc_body(B, T, x_hbm, pos_hbm, val_hbm, out_hbm, posv, valv, buf, win,
             shp, sem_ti, sem_to, sem_si, sem_so, sem_win):
    rows = B // _NW
    nch = T // _CW
    nslab = (rows // 8) * nch
    cid = lax.axis_index("c")
    sid = lax.axis_index("s")
    wid = sid * _NC + cid
    r0 = wid * rows
    pltpu.sync_copy(pos_hbm.at[pl.ds(r0, rows)], posv)
    pltpu.sync_copy(val_hbm.at[pl.ds(r0, rows)], valv)
    lane = lax.broadcasted_iota(jnp.int32, (16,), 0)
    pos16 = [posv[pl.ds(g * 16, 16)] for g in range(rows // 16)]
    val16 = [valv[pl.ds(g * 16, 16)] for g in range(rows // 16)]

    def slab_src(j):
        b, c = divmod(j, nch)
        return x_hbm.at[pl.ds(r0 + b * 8, 8), pl.ds(c * _CW, _CW)]

    def slab_dst(j):
        b, c = divmod(j, nch)
        return out_hbm.at[pl.ds(r0 + b * 8, 8), pl.ds(c * _CW, _CW)]

    def ring(slabs, depth, get_buf, sem_in, sem_out):
        n = len(slabs)
        in_h = [None] * n
        out_h = [None] * n
        for k in range(min(depth - 1, n)):
            in_h[k] = pltpu.async_copy(slab_src(slabs[k]), get_buf(k % depth),
                                       sem_in)
        for j in range(n):
            nxt = j + depth - 1
            if nxt < n:
                if nxt - depth >= 0:
                    out_h[nxt - depth].wait()
                in_h[nxt] = pltpu.async_copy(slab_src(slabs[nxt]),
                                             get_buf(nxt % depth), sem_in)
            in_h[j].wait()
            out_h[j] = pltpu.async_copy(get_buf(j % depth),
                                        slab_dst(slabs[j]), sem_out)
            yield
        for j in range(max(0, n - depth), n):
            out_h[j].wait()

    tile_slabs = [j for j in range(nslab) if j % 2 == 0]
    sp_slabs = [j for j in range(nslab) if j % 2 == 1]
    g_tile = ring(tile_slabs, _DT, lambda s: buf.at[s], sem_ti, sem_to)
    g_sp = ring(sp_slabs, _DS, lambda s: shp.at[sid, s], sem_si, sem_so)

    w0s, offs, vs, gh = [], [], [], []
    for j in range(rows):
        g, l = divmod(j, 16)
        p = pos16[g][l]
        w0 = (p // 16) * 16
        w0s.append(w0)
        offs.append(p - w0)
        vs.append(val16[g][l])
        gh.append(pltpu.async_copy(x_hbm.at[r0 + j, pl.ds(w0, 16)],
                                   win.at[j], sem_win))

    for _ in itertools.zip_longest(g_tile, g_sp):
        pass

    for j in range(rows):
        gh[j].wait()
        win[j] = win[j] + jnp.where(lane == offs[j], vs[j], 0.0)
    sh = [pltpu.async_copy(win.at[j], out_hbm.at[r0 + j, pl.ds(w0s[j], 16)],
                           sem_win)
          for j in range(rows)]
    for h in sh:
        h.wait()


def kernel(x):
    B, T = x.shape
    pos, vals = _spike_consts(B, T, x.dtype)
    mesh = plsc.VectorSubcoreMesh(core_axis_name="c", subcore_axis_name="s",
                                  num_cores=_NC, num_subcores=_NS)
    rows = B // _NW
    sc_call = pl.kernel(
        functools.partial(_sc_body, B, T),
        out_type=jax.ShapeDtypeStruct((B, T), x.dtype),
        mesh=mesh,
        compiler_params=pltpu.CompilerParams(needs_layout_passes=False),
        scratch_types=[
            pltpu.VMEM((rows,), jnp.int32),
            pltpu.VMEM((rows,), jnp.float32),
            pltpu.VMEM((_DT, 8, _CW), jnp.float32),
            pltpu.VMEM((rows, 16), jnp.float32),
            pltpu.VMEM_SHARED((_NS, _DS, 8, _CW), jnp.float32),
            pltpu.SemaphoreType.DMA,
            pltpu.SemaphoreType.DMA,
            pltpu.SemaphoreType.DMA,
            pltpu.SemaphoreType.DMA,
            pltpu.SemaphoreType.DMA,
        ],
    )
    return sc_call(x, pos, vals)
